# Initial kernel scaffold; baseline (speedup 1.0000x reference)
import jax
import jax.numpy as jnp
from jax.experimental import pallas as pl


def kernel(x, edge_index, W1, b1, W2, b2):
    raise NotImplementedError("write your pallas kernel here")


# speedup vs baseline: 4.8189x; 4.8189x over previous
import functools

import jax
import jax.numpy as jnp
from jax import lax
from jax.experimental import pallas as pl
from jax.experimental.pallas import tpu as pltpu
from jax.experimental.pallas import tpu_sc as plsc

N = 10000
E = 320000
D_IN = 128
D_H = 128
D_OUT = 64

NUM_SC = 2
TILES_PER_SC = 16
NW = NUM_SC * TILES_PER_SC
CHUNK = 128
E_PAD = ((E + NW * CHUNK - 1) // (NW * CHUNK)) * (NW * CHUNK)
PER_TILE = E_PAD // NW
N_CHUNKS = PER_TILE // CHUNK
ROWS_PER_TILE = 640
N_ACC = TILES_PER_SC * ROWS_PER_TILE


def _make_sc_agg(D):
    mesh = plsc.VectorSubcoreMesh(core_axis_name="c", subcore_axis_name="s")

    @functools.partial(
        pl.kernel,
        mesh=mesh,
        compiler_params=pltpu.CompilerParams(use_tc_tiling_on_sc=(D == 128)),
        out_type=jax.ShapeDtypeStruct((NUM_SC * N_ACC, D), jnp.float32),
        scratch_types=[
            pltpu.VMEM((CHUNK,), jnp.int32),
            pltpu.VMEM((CHUNK,), jnp.int32),
            pltpu.VMEM((CHUNK, D), jnp.float32),
            pltpu.VMEM((16, D), jnp.float32),
            pltpu.VMEM_SHARED((N_ACC, D), jnp.float32),
            pltpu.SemaphoreType.DMA,
        ],
    )
    def agg(y_hbm, src_hbm, dst_hbm, out_hbm, src_v, dst_v, rows_v, zbuf, acc, sem):
        cid = lax.axis_index("c")
        sid = lax.axis_index("s")
        wid = cid * TILES_PER_SC + sid

        def zrow(i, carry):
            def zcol(j, c2):
                zbuf[i, pl.ds(j * 16, 16)] = jnp.zeros((16,), jnp.float32)
                return c2
            return lax.fori_loop(0, D // 16, zcol, carry)
        lax.fori_loop(0, 16, zrow, 0)

        def zacc(j, carry):
            r = pl.multiple_of(sid * ROWS_PER_TILE + j * 16, 16)
            pltpu.sync_copy(zbuf, acc.at[pl.ds(r, 16)])
            return carry
        lax.fori_loop(0, ROWS_PER_TILE // 16, zacc, 0)

        plsc.subcore_barrier()

        def body(k, carry):
            base = pl.multiple_of(wid * PER_TILE + k * CHUNK, 8)
            pltpu.sync_copy(src_hbm.at[pl.ds(base, CHUNK)], src_v)
            pltpu.sync_copy(dst_hbm.at[pl.ds(base, CHUNK)], dst_v)
            pltpu.async_copy(y_hbm.at[src_v], rows_v, sem).wait()
            pltpu.sync_copy(rows_v, acc.at[dst_v], add=True)
            return carry
        lax.fori_loop(0, N_CHUNKS, body, 0)

        plsc.subcore_barrier()

        r0 = pl.multiple_of(sid * ROWS_PER_TILE, 16)
        pltpu.sync_copy(
            acc.at[pl.ds(r0, ROWS_PER_TILE)],
            out_hbm.at[pl.ds(cid * N_ACC + r0, ROWS_PER_TILE)],
        )

    return agg


_sc_agg_h = _make_sc_agg(D_H)
_sc_agg_o = _make_sc_agg(D_OUT)


def _mm_body(x_ref, w_ref, o_ref):
    o_ref[...] = lax.dot_general(
        x_ref[...], w_ref[...], (((1,), (1,)), ((), ())),
        preferred_element_type=jnp.float32)


def _l2_body(p_ref, b_ref, w_ref, o_ref):
    h = jnp.maximum(p_ref[0] + p_ref[1] + b_ref[...], 0.0)
    o_ref[...] = lax.dot_general(
        h, w_ref[...], (((1,), (1,)), ((), ())),
        preferred_element_type=jnp.float32)


def _l3_body(p_ref, b_ref, o_ref):
    o_ref[...] = p_ref[0] + p_ref[1] + b_ref[...]


_ROWS_BLK = 1000
_N_BLKS = N // _ROWS_BLK


def kernel(x, edge_index, W1, b1, W2, b2):
    src = jnp.concatenate([edge_index[0], jnp.zeros((E_PAD - E,), jnp.int32)])
    dst = jnp.concatenate([edge_index[1], jnp.full((E_PAD - E,), N, jnp.int32)])
    b1r = b1.reshape(1, D_H)
    b2r = b2.reshape(1, D_OUT)

    y1 = pl.pallas_call(
        _mm_body,
        grid=(_N_BLKS,),
        in_specs=[
            pl.BlockSpec((_ROWS_BLK, D_IN), lambda i: (i, 0)),
            pl.BlockSpec((D_H, D_IN), lambda i: (0, 0)),
        ],
        out_specs=pl.BlockSpec((_ROWS_BLK, D_H), lambda i: (i, 0)),
        out_shape=jax.ShapeDtypeStruct((N, D_H), jnp.float32),
    )(x, W1)

    p1 = _sc_agg_h(y1, src, dst).reshape(NUM_SC, N_ACC, D_H)

    y2 = pl.pallas_call(
        _l2_body,
        grid=(_N_BLKS,),
        in_specs=[
            pl.BlockSpec((NUM_SC, _ROWS_BLK, D_H), lambda i: (0, i, 0)),
            pl.BlockSpec((1, D_H), lambda i: (0, 0)),
            pl.BlockSpec((D_OUT, D_H), lambda i: (0, 0)),
        ],
        out_specs=pl.BlockSpec((_ROWS_BLK, D_OUT), lambda i: (i, 0)),
        out_shape=jax.ShapeDtypeStruct((N, D_OUT), jnp.float32),
    )(p1, b1r, W2)

    p2 = _sc_agg_o(y2, src, dst).reshape(NUM_SC, N_ACC, D_OUT)

    out = pl.pallas_call(
        _l3_body,
        grid=(_N_BLKS,),
        in_specs=[
            pl.BlockSpec((NUM_SC, _ROWS_BLK, D_OUT), lambda i: (0, i, 0)),
            pl.BlockSpec((1, D_OUT), lambda i: (0, 0)),
        ],
        out_specs=pl.BlockSpec((_ROWS_BLK, D_OUT), lambda i: (i, 0)),
        out_shape=jax.ShapeDtypeStruct((N, D_OUT), jnp.float32),
    )(p2, b2r)

    return out
